# baseline (speedup 1.0000x reference)
import functools

import jax
import jax.numpy as jnp
import numpy as np
from jax import lax
from jax.experimental import pallas as pl
from jax.experimental.pallas import tpu as pltpu
from jax.experimental.pallas import tpu_sc as plsc

NC = 2
NS = 16
NW = NC * NS
LANES = 16
T = 32
NBUF = 4
NRES = 16


def _sc_gather(struct_idx, res_idx, struct_w, res_wp):
    NWK = struct_idx.shape[0]
    CH = struct_idx.shape[1] // T
    D = struct_w.shape[1]
    mesh = plsc.VectorSubcoreMesh(core_axis_name="c", subcore_axis_name="s",
                                  num_cores=NC, num_subcores=NS)

    @functools.partial(
        pl.kernel,
        out_type=jax.ShapeDtypeStruct((NWK, CH, T, D), jnp.float32),
        mesh=mesh,
        scratch_types=[
            pltpu.VMEM((CH * T, 1), jnp.int32),
            pltpu.VMEM((CH * T, NRES), jnp.int32),
            pltpu.VMEM((2, T, D), jnp.float32),
            pltpu.VMEM((NBUF, NRES, D // 2), jnp.int32),
            pltpu.VMEM((NBUF, 1, D), jnp.float32),
            pltpu.SemaphoreType.DMA,
            pltpu.SemaphoreType.DMA,
            pltpu.SemaphoreType.DMA,
            pltpu.SemaphoreType.DMA,
            pltpu.SemaphoreType.DMA,
        ],
    )
    def sck(sidx_hbm, ridx_hbm, sw_hbm, rw_hbm, out_hbm,
            sidx_v, ridx_v, acc_v, rbuf_v, sbuf_v, sem0, sem1, sem2, sem3,
            wsem):
        rsem = [sem0, sem1, sem2, sem3]
        wid = lax.axis_index("s") * NC + lax.axis_index("c")
        pltpu.sync_copy(sidx_hbm.at[wid], sidx_v)
        pltpu.sync_copy(ridx_hbm.at[wid], ridx_v)

        def chunk(c, carry):
            p = lax.rem(c, 2)

            def fire(t):
                g = c * T + t
                q = t % NBUF
                return (pltpu.async_copy(rw_hbm.at[ridx_v.at[g]],
                                         rbuf_v.at[q], rsem[q]),
                        pltpu.async_copy(sw_hbm.at[sidx_v.at[g]],
                                         sbuf_v.at[q], rsem[q]))

            dmas = {}
            for t in range(min(NBUF - 1, T)):
                dmas[t] = fire(t)

            @pl.when(c >= 2)
            def _():
                pltpu.make_async_copy(
                    acc_v.at[p], out_hbm.at[wid, c - 2], wsem).wait()

            for t in range(T):
                q = t % NBUF
                dmas[t][0].wait()
                dmas[t][1].wait()
                nt = t + NBUF - 1
                if nt < T:
                    dmas[nt] = fire(nt)

                def body(gi, cr, q=q, t=t):
                    base = gi * (2 * LANES)
                    sa = [sbuf_v[q, 0, pl.ds(base, LANES)], None]
                    sb = [sbuf_v[q, 0, pl.ds(base + LANES, LANES)], None]
                    for n in range(NRES):
                        xi = rbuf_v[q, n, pl.ds(gi * LANES, LANES)]
                        a = lax.bitcast_convert_type(xi << 16, jnp.float32)
                        b = lax.bitcast_convert_type(xi, jnp.float32)
                        k = n & 1
                        sa[k] = a if sa[k] is None else sa[k] + a
                        sb[k] = b if sb[k] is None else sb[k] + b
                    acc_v[p, t, pl.ds(base, LANES)] = sa[0] + sa[1]
                    acc_v[p, t, pl.ds(base + LANES, LANES)] = sb[0] + sb[1]
                    return cr

                lax.fori_loop(0, D // (2 * LANES), body, 0)
            pltpu.async_copy(acc_v.at[p], out_hbm.at[wid, c], wsem)
            return carry

        lax.fori_loop(0, CH, chunk, 0)
        pltpu.make_async_copy(
            acc_v.at[(CH - 2) % 2], out_hbm.at[wid, CH - 2], wsem).wait()
        pltpu.make_async_copy(
            acc_v.at[(CH - 1) % 2], out_hbm.at[wid, CH - 1], wsem).wait()

    return sck(struct_idx, res_idx, struct_w, res_wp)


def _tc_dense(gath_sc, struct_tok, seq_tok, ss8_tok, sasa_tok, func_tok,
              res_tok, avg, per, seq_w, plddt_W, plddt_b, spp_W, spp_b,
              ss8_w, sasa_w, func_w, res_w0, struct_w, res_w):
    TOKENS = seq_tok.shape[0]
    SC_TOK, D = gath_sc.shape
    TB = 512
    NSCB = SC_TOK // TB
    grid = (TOKENS // TB,)
    f32 = jnp.float32
    hp = lax.Precision.HIGHEST
    NBINS = plddt_W.shape[0]
    KS = struct_w.shape[0]
    KR = res_w.shape[0]

    def body(gath_ref, st_ref, seq_ref, ss8_ref, sasa_ref, func_ref, res_ref,
             avg_ref, per_ref, seqw_ref, pw_ref, pb_ref, sw_ref, sb_ref,
             ss8w_ref, sasaw_ref, funcw_ref, rw0_ref, sw2_ref, rw_ref,
             out_ref):
        lp = lax.Precision.DEFAULT

        def onehot_mm(idx_col, k, w):
            iot = lax.broadcasted_iota(jnp.int32, (TB, k), 1)
            oh = (idx_col == iot).astype(f32)
            return jnp.dot(oh, w, precision=lp, preferred_element_type=f32)

        acc = onehot_mm(seq_ref[:], seqw_ref.shape[0], seqw_ref[:])
        acc += onehot_mm(ss8_ref[:], ss8w_ref.shape[0], ss8w_ref[:])
        acc += onehot_mm(sasa_ref[:], sasaw_ref.shape[0], sasaw_ref[:])

        inv_std = float(NBINS)

        def rbf_proj(v_ref, w_ref, b_ref):
            ci = lax.broadcasted_iota(jnp.int32, (1, NBINS), 1)
            centers = ci.astype(f32) * np.float32(1.0 / (NBINS - 1))
            z = (v_ref[:] - centers) * inv_std
            feats = jnp.exp(-(z * z))
            return jnp.dot(feats, w_ref[:], precision=hp,
                           preferred_element_type=f32) + b_ref[:]

        acc += rbf_proj(avg_ref, pw_ref, pb_ref)
        acc += rbf_proj(per_ref, sw_ref, sb_ref)

        parts = []
        kf = funcw_ref.shape[1]
        for i in range(funcw_ref.shape[0]):
            idx = func_ref[:, i:i + 1]
            iot = lax.broadcasted_iota(jnp.int32, (TB, kf), 1)
            oh = jnp.logical_and(idx == iot, idx != 0).astype(f32)
            parts.append(jnp.dot(oh, funcw_ref[i], precision=lp,
                                 preferred_element_type=f32))
        acc += jnp.concatenate(parts, axis=-1)

        cnt0 = jnp.sum((res_ref[:] == 0).astype(f32), axis=1, keepdims=True)
        acc -= cnt0 * rw0_ref[:]

        blk = pl.program_id(0)

        @pl.when(blk < NSCB)
        def _():
            out_ref[:] = acc + gath_ref[:]

        @pl.when(blk >= NSCB)
        def _():
            iot = lax.broadcasted_iota(jnp.int32, (TB, KS), 1)
            oh = (st_ref[:] == iot).astype(f32)
            g = jnp.dot(oh, sw2_ref[:], precision=lp,
                        preferred_element_type=f32)
            iotr = lax.broadcasted_iota(jnp.int32, (TB, KR), 1)
            cnt = None
            for n in range(NRES):
                c1 = (res_ref[:, n:n + 1] == iotr).astype(f32)
                cnt = c1 if cnt is None else cnt + c1
            g = g + jnp.dot(cnt, rw_ref[:], precision=lp,
                            preferred_element_type=f32)
            out_ref[:] = acc + g

    tok_spec = lambda w: pl.BlockSpec((TB, w), lambda i: (i, 0))
    full2 = lambda a: pl.BlockSpec(a.shape, lambda i: (0, 0))
    return pl.pallas_call(
        body,
        grid=grid,
        in_specs=[
            pl.BlockSpec((TB, D), lambda i: (jnp.minimum(i, NSCB - 1), 0)),
            tok_spec(1), tok_spec(1), tok_spec(1), tok_spec(1),
            tok_spec(func_tok.shape[1]), tok_spec(NRES), tok_spec(1),
            tok_spec(1),
            full2(seq_w), full2(plddt_W), full2(plddt_b), full2(spp_W),
            full2(spp_b), full2(ss8_w), full2(sasa_w),
            pl.BlockSpec(func_w.shape, lambda i: (0, 0, 0)),
            full2(res_w0), full2(struct_w), full2(res_w),
        ],
        out_specs=tok_spec(D),
        out_shape=jax.ShapeDtypeStruct((TOKENS, D), f32),
    )(gath_sc, struct_tok, seq_tok, ss8_tok, sasa_tok, func_tok, res_tok,
      avg, per, seq_w, plddt_W, plddt_b, spp_W, spp_b, ss8_w, sasa_w,
      func_w, res_w0, struct_w, res_w)


def kernel(sequence_tokens, structure_tokens, average_plddt, per_res_plddt,
           ss8_tokens, sasa_tokens, function_tokens, residue_annotation_tokens,
           seq_w, plddt_W, plddt_b, spp_W, spp_b, struct_w, ss8_w, sasa_w,
           func_w, res_w):
    B, L = sequence_tokens.shape
    TOKENS = B * L
    D = seq_w.shape[1]
    SC_TOK = (TOKENS // 2) // (NW * T) * (NW * T)
    CH = SC_TOK // (NW * T)
    i32 = jnp.int32

    st_flat = structure_tokens.reshape(TOKENS).astype(i32)
    rt_flat = residue_annotation_tokens.reshape(TOKENS, NRES).astype(i32)
    sidx = st_flat[:SC_TOK].reshape(NW, CH * T, 1)
    ridx = rt_flat[:SC_TOK].reshape(NW, CH * T, NRES)
    res_wp = jax.lax.bitcast_convert_type(
        res_w.astype(jnp.bfloat16).reshape(-1, D // 32, 2, LANES)
        .transpose(0, 1, 3, 2),
        jnp.int32).reshape(-1, D // 2)
    gath_sc = _sc_gather(sidx, ridx, struct_w, res_wp).reshape(SC_TOK, D)

    out = _tc_dense(
        gath_sc,
        st_flat.reshape(TOKENS, 1),
        sequence_tokens.reshape(TOKENS, 1).astype(i32),
        ss8_tokens.reshape(TOKENS, 1).astype(i32),
        sasa_tokens.reshape(TOKENS, 1).astype(i32),
        function_tokens.reshape(TOKENS, -1).astype(i32),
        residue_annotation_tokens.reshape(TOKENS, NRES).astype(i32),
        average_plddt.reshape(TOKENS, 1).astype(jnp.float32),
        per_res_plddt.reshape(TOKENS, 1).astype(jnp.float32),
        seq_w, plddt_W, plddt_b.reshape(1, D), spp_W, spp_b.reshape(1, D),
        ss8_w, sasa_w, func_w, res_w[0].reshape(1, D), struct_w, res_w)
    return out.reshape(B, L, D)


# speedup vs baseline: 1.1404x; 1.1404x over previous
import functools

import jax
import jax.numpy as jnp
import numpy as np
from jax import lax
from jax.experimental import pallas as pl
from jax.experimental.pallas import tpu as pltpu
from jax.experimental.pallas import tpu_sc as plsc

NC = 2
NS = 16
NW = NC * NS
LANES = 16
T = 16
NBUF = 4
NRES = 16


def _sc_gather(struct_idx, res_idx, struct_w, res_wp):
    NWK, CH, _ = struct_idx.shape
    D = struct_w.shape[1]
    mesh = plsc.VectorSubcoreMesh(core_axis_name="c", subcore_axis_name="s",
                                  num_cores=NC, num_subcores=NS)

    @functools.partial(
        pl.kernel,
        out_type=jax.ShapeDtypeStruct((NWK, CH, T, D), jnp.float32),
        mesh=mesh,
        scratch_types=[
            pltpu.VMEM((CH, T), jnp.int32),
            pltpu.VMEM((CH * T, NRES), jnp.int32),
            pltpu.VMEM((2, T, D), jnp.float32),
            pltpu.VMEM((NBUF, NRES, D // 2), jnp.int32),
            pltpu.SemaphoreType.DMA,
            pltpu.SemaphoreType.DMA,
            pltpu.SemaphoreType.DMA,
            pltpu.SemaphoreType.DMA,
            pltpu.SemaphoreType.DMA,
        ],
    )
    def sck(sidx_hbm, ridx_hbm, sw_hbm, rw_hbm, out_hbm,
            sidx_v, ridx_v, acc_v, rbuf_v, sem0, sem1, sem2, sem3, wsem):
        rsem = [sem0, sem1, sem2, sem3]
        wid = lax.axis_index("s") * NC + lax.axis_index("c")
        pltpu.sync_copy(sidx_hbm.at[wid], sidx_v)
        pltpu.sync_copy(ridx_hbm.at[wid], ridx_v)

        def chunk(c, carry):
            p = lax.rem(c, 2)
            dmas = {}
            for t in range(min(NBUF - 1, T)):
                dmas[t] = pltpu.async_copy(
                    rw_hbm.at[ridx_v.at[c * T + t]], rbuf_v.at[t % NBUF],
                    rsem[t % NBUF])

            @pl.when(c >= 2)
            def _():
                pltpu.make_async_copy(
                    acc_v.at[p], out_hbm.at[wid, c - 2], wsem).wait()

            pltpu.sync_copy(sw_hbm.at[sidx_v.at[c]], acc_v.at[p])
            for t in range(T):
                q = t % NBUF
                dmas[t].wait()
                nt = t + NBUF - 1
                if nt < T:
                    dmas[nt] = pltpu.async_copy(
                        rw_hbm.at[ridx_v.at[c * T + nt]], rbuf_v.at[nt % NBUF],
                        rsem[nt % NBUF])

                def body(gi, cr, q=q, t=t):
                    base = gi * (2 * LANES)
                    sa = [None, None]
                    sb = [None, None]
                    for n in range(NRES):
                        xi = rbuf_v[q, n, pl.ds(gi * LANES, LANES)]
                        a = lax.bitcast_convert_type(xi << 16, jnp.float32)
                        b = lax.bitcast_convert_type(xi, jnp.float32)
                        k = n & 1
                        sa[k] = a if sa[k] is None else sa[k] + a
                        sb[k] = b if sb[k] is None else sb[k] + b
                    plsc.addupdate(acc_v.at[p, t, pl.ds(base, LANES)],
                                   sa[0] + sa[1])
                    plsc.addupdate(acc_v.at[p, t, pl.ds(base + LANES, LANES)],
                                   sb[0] + sb[1])
                    return cr

                lax.fori_loop(0, D // (2 * LANES), body, 0)
            pltpu.async_copy(acc_v.at[p], out_hbm.at[wid, c], wsem)
            return carry

        lax.fori_loop(0, CH, chunk, 0)
        pltpu.make_async_copy(
            acc_v.at[(CH - 2) % 2], out_hbm.at[wid, CH - 2], wsem).wait()
        pltpu.make_async_copy(
            acc_v.at[(CH - 1) % 2], out_hbm.at[wid, CH - 1], wsem).wait()

    return sck(struct_idx, res_idx, struct_w, res_wp)


def _tc_gath(struct_tok, res_tok, struct_w, res_w):
    N = struct_tok.shape[0]
    KS, D = struct_w.shape
    KR = res_w.shape[0]
    TBg = 256
    f32 = jnp.float32
    lp = lax.Precision.DEFAULT

    def body(st_ref, rt_ref, sw_ref, rw_ref, out_ref):
        iot = lax.broadcasted_iota(jnp.int32, (TBg, KS), 1)
        oh = (st_ref[:] == iot).astype(f32)
        acc = jnp.dot(oh, sw_ref[:], precision=lp, preferred_element_type=f32)
        iotr = lax.broadcasted_iota(jnp.int32, (TBg, KR), 1)
        cnt = None
        for n in range(NRES):
            c1 = (rt_ref[:, n:n + 1] == iotr).astype(f32)
            cnt = c1 if cnt is None else cnt + c1
        acc = acc + jnp.dot(cnt, rw_ref[:], precision=lp,
                            preferred_element_type=f32)
        out_ref[:] = acc

    return pl.pallas_call(
        body,
        grid=(N // TBg,),
        in_specs=[
            pl.BlockSpec((TBg, 1), lambda i: (i, 0)),
            pl.BlockSpec((TBg, NRES), lambda i: (i, 0)),
            pl.BlockSpec(struct_w.shape, lambda i: (0, 0)),
            pl.BlockSpec(res_w.shape, lambda i: (0, 0)),
        ],
        out_specs=pl.BlockSpec((TBg, D), lambda i: (i, 0)),
        out_shape=jax.ShapeDtypeStruct((N, D), f32),
    )(struct_tok, res_tok, struct_w, res_w)


def _tc_dense(gath, seq_tok, ss8_tok, sasa_tok, func_tok, res_tok, avg, per,
              seq_w, plddt_W, plddt_b, spp_W, spp_b, ss8_w, sasa_w, func_w,
              res_w0):
    TOKENS, D = gath.shape
    TB = 512
    grid = (TOKENS // TB,)
    f32 = jnp.float32
    hp = lax.Precision.HIGHEST
    NBINS = plddt_W.shape[0]

    def body(gath_ref, seq_ref, ss8_ref, sasa_ref, func_ref, res_ref, avg_ref,
             per_ref, seqw_ref, pw_ref, pb_ref, sw_ref, sb_ref, ss8w_ref,
             sasaw_ref, funcw_ref, rw0_ref, out_ref):
        lp = lax.Precision.DEFAULT

        def onehot_mm(idx_col, k, w):
            iot = lax.broadcasted_iota(jnp.int32, (TB, k), 1)
            oh = (idx_col == iot).astype(f32)
            return jnp.dot(oh, w, precision=lp, preferred_element_type=f32)

        acc = gath_ref[:]
        acc += onehot_mm(seq_ref[:], seqw_ref.shape[0], seqw_ref[:])
        acc += onehot_mm(ss8_ref[:], ss8w_ref.shape[0], ss8w_ref[:])
        acc += onehot_mm(sasa_ref[:], sasaw_ref.shape[0], sasaw_ref[:])

        inv_std = float(NBINS)

        def rbf_proj(v_ref, w_ref, b_ref):
            ci = lax.broadcasted_iota(jnp.int32, (1, NBINS), 1)
            centers = ci.astype(f32) * np.float32(1.0 / (NBINS - 1))
            z = (v_ref[:] - centers) * inv_std
            feats = jnp.exp(-(z * z))
            return jnp.dot(feats, w_ref[:], precision=hp,
                           preferred_element_type=f32) + b_ref[:]

        acc += rbf_proj(avg_ref, pw_ref, pb_ref)
        acc += rbf_proj(per_ref, sw_ref, sb_ref)

        parts = []
        kf = funcw_ref.shape[1]
        for i in range(funcw_ref.shape[0]):
            idx = func_ref[:, i:i + 1]
            iot = lax.broadcasted_iota(jnp.int32, (TB, kf), 1)
            oh = jnp.logical_and(idx == iot, idx != 0).astype(f32)
            parts.append(jnp.dot(oh, funcw_ref[i], precision=lp,
                                 preferred_element_type=f32))
        acc += jnp.concatenate(parts, axis=-1)

        cnt = jnp.sum((res_ref[:] == 0).astype(f32), axis=1, keepdims=True)
        acc -= cnt * rw0_ref[:]
        out_ref[:] = acc

    tok_spec = lambda w: pl.BlockSpec((TB, w), lambda i: (i, 0))
    full2 = lambda a: pl.BlockSpec(a.shape, lambda i: (0, 0))
    return pl.pallas_call(
        body,
        grid=grid,
        in_specs=[
            tok_spec(D), tok_spec(1), tok_spec(1), tok_spec(1),
            tok_spec(func_tok.shape[1]), tok_spec(NRES), tok_spec(1),
            tok_spec(1),
            full2(seq_w), full2(plddt_W), full2(plddt_b), full2(spp_W),
            full2(spp_b), full2(ss8_w), full2(sasa_w),
            pl.BlockSpec(func_w.shape, lambda i: (0, 0, 0)),
            full2(res_w0),
        ],
        out_specs=tok_spec(D),
        out_shape=jax.ShapeDtypeStruct((TOKENS, D), f32),
    )(gath, seq_tok, ss8_tok, sasa_tok, func_tok, res_tok, avg, per,
      seq_w, plddt_W, plddt_b, spp_W, spp_b, ss8_w, sasa_w, func_w, res_w0)


def kernel(sequence_tokens, structure_tokens, average_plddt, per_res_plddt,
           ss8_tokens, sasa_tokens, function_tokens, residue_annotation_tokens,
           seq_w, plddt_W, plddt_b, spp_W, spp_b, struct_w, ss8_w, sasa_w,
           func_w, res_w):
    B, L = sequence_tokens.shape
    TOKENS = B * L
    D = seq_w.shape[1]
    SC_TOK = (TOKENS * 5 // 8) // (NW * T) * (NW * T)
    CH = SC_TOK // (NW * T)
    i32 = jnp.int32

    st_flat = structure_tokens.reshape(TOKENS).astype(i32)
    rt_flat = residue_annotation_tokens.reshape(TOKENS, NRES).astype(i32)
    sidx = st_flat[:SC_TOK].reshape(NW, CH, T)
    ridx = rt_flat[:SC_TOK].reshape(NW, CH * T, NRES)
    res_wp = jax.lax.bitcast_convert_type(
        res_w.astype(jnp.bfloat16).reshape(-1, D // 32, 2, LANES)
        .transpose(0, 1, 3, 2),
        jnp.int32).reshape(-1, D // 2)
    gath_sc = _sc_gather(sidx, ridx, struct_w, res_wp).reshape(SC_TOK, D)
    gath_tc = _tc_gath(st_flat[SC_TOK:].reshape(-1, 1), rt_flat[SC_TOK:],
                       struct_w, res_w)
    gath = jnp.concatenate([gath_sc, gath_tc], axis=0)

    out = _tc_dense(
        gath,
        sequence_tokens.reshape(TOKENS, 1).astype(i32),
        ss8_tokens.reshape(TOKENS, 1).astype(i32),
        sasa_tokens.reshape(TOKENS, 1).astype(i32),
        function_tokens.reshape(TOKENS, -1).astype(i32),
        residue_annotation_tokens.reshape(TOKENS, NRES).astype(i32),
        average_plddt.reshape(TOKENS, 1).astype(jnp.float32),
        per_res_plddt.reshape(TOKENS, 1).astype(jnp.float32),
        seq_w, plddt_W, plddt_b.reshape(1, D), spp_W, spp_b.reshape(1, D),
        ss8_w, sasa_w, func_w, res_w[0].reshape(1, D))
    return out.reshape(B, L, D)
